# speedup vs baseline: 1.1231x; 1.1231x over previous
import functools

import jax
import jax.numpy as jnp
from jax import lax
from jax.experimental import pallas as pl
from jax.experimental.pallas import tpu as pltpu

_B, _S, _K, _D = 2, 2048, 2, 1024
_E = 8
_TOPK = 2
_DFF = 2048
_N = _B * _S * _K
_R = _N * _TOPK
_TM = 256
_P = _R + _E * _TM
_NT = _P // _TM

_NEG = -3.0e38


def _gate_body(x_ref, gw_ref, out_ref):
    logits = jnp.dot(x_ref[...], gw_ref[...], preferred_element_type=jnp.float32)
    rows = logits.shape[0]
    col = lax.broadcasted_iota(jnp.int32, (rows, _E), 1)
    m1 = jnp.max(logits, axis=1, keepdims=True)
    i1 = jnp.min(jnp.where(logits == m1, col, _E), axis=1, keepdims=True)
    masked = jnp.where(col == i1, _NEG, logits)
    m2 = jnp.max(masked, axis=1, keepdims=True)
    i2 = jnp.min(jnp.where(masked == m2, col, _E), axis=1, keepdims=True)
    w1st = 1.0 / (1.0 + jnp.exp(m2 - m1))
    w2nd = 1.0 - w1st
    out_ref[:, 0:1] = i1.astype(jnp.float32)
    out_ref[:, 1:2] = i2.astype(jnp.float32)
    out_ref[:, 2:3] = w1st
    out_ref[:, 3:4] = w2nd


def _gating(x2d, gate_w):
    blk = 1024
    return pl.pallas_call(
        _gate_body,
        grid=(_N // blk,),
        in_specs=[
            pl.BlockSpec((blk, _D), lambda i: (i, 0)),
            pl.BlockSpec((_D, _E), lambda i: (0, 0)),
        ],
        out_specs=pl.BlockSpec((blk, 4), lambda i: (i, 0)),
        out_shape=jax.ShapeDtypeStruct((_N, 4), jnp.float32),
    )(x2d, gate_w)


def _mm_body(te_ref, xg_ref, w1_ref, w2_ref, y_ref):
    h = jnp.dot(xg_ref[...], w1_ref[0], preferred_element_type=jnp.float32)
    h = h * jax.nn.sigmoid(h)
    y_ref[...] = jnp.dot(h, w2_ref[0], preferred_element_type=jnp.float32)


def _grouped_mm(xg, w1, w2, tile_expert):
    grid_spec = pltpu.PrefetchScalarGridSpec(
        num_scalar_prefetch=1,
        grid=(_NT,),
        in_specs=[
            pl.BlockSpec((_TM, _D), lambda t, te: (t, 0)),
            pl.BlockSpec((1, _D, _DFF), lambda t, te: (te[t], 0, 0)),
            pl.BlockSpec((1, _DFF, _D), lambda t, te: (te[t], 0, 0)),
        ],
        out_specs=pl.BlockSpec((_TM, _D), lambda t, te: (t, 0)),
    )
    return pl.pallas_call(
        _mm_body,
        grid_spec=grid_spec,
        out_shape=jax.ShapeDtypeStruct((_P, _D), jnp.float32),
    )(tile_expert, xg, w1, w2)


def _route(ids_flat):
    e_arange = jnp.arange(_E, dtype=jnp.int32)
    counts = jnp.sum(
        (ids_flat[:, None] == e_arange[None, :]).astype(jnp.int32), axis=0
    )
    cnt_pad = ((counts + _TM - 1) // _TM) * _TM
    cum_pad = jnp.cumsum(cnt_pad)
    start_pad = cum_pad - cnt_pad
    cum = jnp.cumsum(counts)
    starts = cum - counts
    order = jnp.argsort(ids_flat, stable=True)
    e_sorted = ids_flat[order]
    j = jnp.arange(_R, dtype=jnp.int32)
    pos = start_pad[e_sorted] + (j - starts[e_sorted])
    row_token = jnp.zeros((_P,), jnp.int32).at[pos].set(
        (order // _TOPK).astype(jnp.int32)
    )
    inv = jnp.zeros((_R,), jnp.int32).at[order].set(pos)
    tile_base = jnp.arange(_NT, dtype=jnp.int32) * _TM
    tile_expert = jnp.clip(
        jnp.searchsorted(cum_pad, tile_base, side="right"), 0, _E - 1
    ).astype(jnp.int32)
    return row_token, inv, tile_expert


def kernel(inputs, gate_w, w1, w2):
    x2d = inputs.reshape(_N, _D)
    g = _gating(x2d, gate_w)
    ids = g[:, :2].astype(jnp.int32)
    wts = g[:, 2:4]
    row_token, inv, tile_expert = _route(ids.reshape(-1))
    xg = x2d[row_token]
    y = _grouped_mm(xg, w1, w2, tile_expert)
    inv2 = inv.reshape(_N, _TOPK)
    out = wts[:, 0:1] * y[inv2[:, 0]] + wts[:, 1:2] * y[inv2[:, 1]]
    return out.reshape(_B, _S, _K, _D)
